# Initial kernel scaffold; baseline (speedup 1.0000x reference)
import functools
import math

import jax
import jax.numpy as jnp
from jax.experimental import pallas as pl
from jax.experimental.pallas import tpu as pltpu
from jax.experimental.pallas import tpu_sc as plsc

S = 2048
D = 1024
H = 16
DH = 64
E = 64
TOPK = 2
CAP = int(math.ceil(S * TOPK / E * 1.25))
EPS = 1e-5
NSLOT = E * CAP
NPAIR = S * TOPK

f32 = jnp.float32
bf16 = jnp.bfloat16
i32 = jnp.int32

SC_NC = 2
SC_NS = 16
SC_NW = SC_NC * SC_NS


def _mm_body(x_ref, w_ref, b_ref, o_ref):
    acc = jnp.dot(x_ref[...], w_ref[...], preferred_element_type=f32)
    o_ref[...] = (acc + b_ref[...]).astype(o_ref.dtype)


def _matmul_bias(x_bf, w_bf, b, out_dtype, bm=256):
    m, k = x_bf.shape
    n = w_bf.shape[1]
    return pl.pallas_call(
        _mm_body,
        grid=(m // bm,),
        in_specs=[
            pl.BlockSpec((bm, k), lambda i: (i, 0)),
            pl.BlockSpec((k, n), lambda i: (0, 0)),
            pl.BlockSpec((1, n), lambda i: (0, 0)),
        ],
        out_specs=pl.BlockSpec((bm, n), lambda i: (i, 0)),
        out_shape=jax.ShapeDtypeStruct((m, n), out_dtype),
    )(x_bf, w_bf, b.reshape(1, n))


def _attn_body(q_ref, k_ref, v_ref, o_ref):
    q = q_ref[0]
    k = k_ref[0]
    s = jax.lax.dot_general(q, k, (((1,), (1,)), ((), ())),
                            preferred_element_type=f32) * (1.0 / 8.0)
    m = jnp.max(s, axis=-1, keepdims=True)
    p = jnp.exp(s - m)
    p = p / jnp.sum(p, axis=-1, keepdims=True)
    o = jnp.dot(p.astype(bf16), v_ref[0], preferred_element_type=f32)
    o_ref[0] = o.astype(bf16)


def _flash_attn(q_hm, k_hm, v_hm, bq=256):
    return pl.pallas_call(
        _attn_body,
        grid=(H, S // bq),
        in_specs=[
            pl.BlockSpec((1, bq, DH), lambda h, qb: (h, qb, 0)),
            pl.BlockSpec((1, S, DH), lambda h, qb: (h, 0, 0)),
            pl.BlockSpec((1, S, DH), lambda h, qb: (h, 0, 0)),
        ],
        out_specs=pl.BlockSpec((1, bq, DH), lambda h, qb: (h, qb, 0)),
        out_shape=jax.ShapeDtypeStruct((H, S, DH), bf16),
    )(q_hm, k_hm, v_hm)


def _proj_ln_body(a_ref, w_ref, b_ref, res_ref, g_ref, bb_ref, of_ref, ob_ref):
    y = jnp.dot(a_ref[...], w_ref[...], preferred_element_type=f32)
    y = y + b_ref[...] + res_ref[...]
    mu = jnp.mean(y, axis=-1, keepdims=True)
    var = jnp.mean((y - mu) ** 2, axis=-1, keepdims=True)
    z = (y - mu) / jnp.sqrt(var + EPS) * g_ref[...] + bb_ref[...]
    of_ref[...] = z
    ob_ref[...] = z.astype(bf16)


def _proj_res_ln(a_bf, w_bf, b, res, g, bb, bm=256):
    return pl.pallas_call(
        _proj_ln_body,
        grid=(S // bm,),
        in_specs=[
            pl.BlockSpec((bm, D), lambda i: (i, 0)),
            pl.BlockSpec((D, D), lambda i: (0, 0)),
            pl.BlockSpec((1, D), lambda i: (0, 0)),
            pl.BlockSpec((bm, D), lambda i: (i, 0)),
            pl.BlockSpec((1, D), lambda i: (0, 0)),
            pl.BlockSpec((1, D), lambda i: (0, 0)),
        ],
        out_specs=(
            pl.BlockSpec((bm, D), lambda i: (i, 0)),
            pl.BlockSpec((bm, D), lambda i: (i, 0)),
        ),
        out_shape=(
            jax.ShapeDtypeStruct((S, D), f32),
            jax.ShapeDtypeStruct((S, D), bf16),
        ),
    )(a_bf, w_bf, b.reshape(1, D), res, g.reshape(1, D), bb.reshape(1, D))


def _route_body(x_ref, w_ref, b_ref, slot_ref, pw_ref, src_ref):
    logits = jnp.dot(x_ref[...], w_ref[...], preferred_element_type=f32)
    logits = logits + b_ref[...]
    mx = jnp.max(logits, axis=-1, keepdims=True)
    ex = jnp.exp(logits - mx)
    probs = ex / jnp.sum(ex, axis=-1, keepdims=True)

    cols = jax.lax.broadcasted_iota(i32, (S, E), 1)
    m1 = jnp.max(probs, axis=-1, keepdims=True)
    i1 = jnp.min(jnp.where(probs == m1, cols, E), axis=-1, keepdims=True)
    probs_m = jnp.where(cols == i1, -jnp.float32(1e30), probs)
    m2 = jnp.max(probs_m, axis=-1, keepdims=True)
    i2 = jnp.min(jnp.where(probs_m == m2, cols, E), axis=-1, keepdims=True)
    gsum = m1 + m2
    g1 = m1 / gsum
    g2 = m2 / gsum

    ohE = (cols == i1).astype(f32)
    ohO = (cols == i2).astype(f32)

    cb = 256
    r_i = jax.lax.broadcasted_iota(i32, (cb, cb), 0)
    c_i = jax.lax.broadcasted_iota(i32, (cb, cb), 1)
    lt = (r_i >= c_i).astype(f32)
    dnum = (((0,), (0,)), ((), ()))
    pos0_parts = []
    pos1_parts = []
    carryE = jnp.zeros((1, E), f32)
    carryO = jnp.zeros((1, E), f32)
    for ci in range(S // cb):
        sl = slice(ci * cb, (ci + 1) * cb)
        ohE_c = ohE[sl]
        ohO_c = ohO[sl]
        ce = jnp.dot(lt, ohE_c, preferred_element_type=f32) + carryE
        co = jnp.dot(lt, ohO_c, preferred_element_type=f32) + carryO
        tot = ce + co
        pos0_parts.append(
            jnp.sum((tot - ohO_c) * ohE_c, axis=-1, keepdims=True) - 1.0)
        pos1_parts.append(jnp.sum(tot * ohO_c, axis=-1, keepdims=True) - 1.0)
        carryE = ce[cb - 1:cb, :]
        carryO = co[cb - 1:cb, :]
    pos0 = jnp.concatenate(pos0_parts, axis=0)
    pos1 = jnp.concatenate(pos1_parts, axis=0)

    keep0 = (pos0 < CAP).astype(f32)
    keep1 = (pos1 < CAP).astype(f32)
    posc0 = jnp.minimum(pos0, CAP - 1.0)
    posc1 = jnp.minimum(pos1, CAP - 1.0)

    slot_ref[:, 0:1] = i1 * CAP + posc0.astype(i32)
    slot_ref[:, 1:2] = i2 * CAP + posc1.astype(i32)
    pw_ref[:, 0:1] = g1 * keep0
    pw_ref[:, 1:2] = g2 * keep1

    ccap = jax.lax.broadcasted_iota(f32, (S, CAP), 1)
    t_f = jax.lax.broadcasted_iota(f32, (S, 1), 0)
    th = jnp.floor(t_f / 32.0)
    tl = t_f - 32.0 * th
    src = jnp.zeros((E, CAP), f32)
    for oh_k, posc_k, keep_k in ((ohE, posc0, keep0), (ohO, posc1, keep1)):
        ohc = (ccap == posc_k).astype(f32) * keep_k
        src = src + 32.0 * jax.lax.dot_general(
            oh_k, ohc * th, dnum, preferred_element_type=f32)
        src = src + jax.lax.dot_general(
            oh_k, ohc * tl, dnum, preferred_element_type=f32)
    src_ref[...] = src.astype(i32)


def _route_dispatch(x_bf, rw_bf, rb):
    return pl.pallas_call(
        _route_body,
        in_specs=[
            pl.BlockSpec((S, D), lambda: (0, 0)),
            pl.BlockSpec((D, E), lambda: (0, 0)),
            pl.BlockSpec((1, E), lambda: (0, 0)),
        ],
        out_specs=(
            pl.BlockSpec((S, TOPK), lambda: (0, 0)),
            pl.BlockSpec((S, TOPK), lambda: (0, 0)),
            pl.BlockSpec((E, CAP), lambda: (0, 0)),
        ),
        out_shape=(
            jax.ShapeDtypeStruct((S, TOPK), i32),
            jax.ShapeDtypeStruct((S, TOPK), f32),
            jax.ShapeDtypeStruct((E, CAP), i32),
        ),
    )(x_bf, rw_bf, rb.reshape(1, E))


def _sc_gather(table, idx, nrows, chunk):
    b_per_w = nrows // SC_NW
    nch = b_per_w // chunk
    mesh = plsc.VectorSubcoreMesh(core_axis_name="c", subcore_axis_name="s")

    @functools.partial(
        pl.kernel,
        mesh=mesh,
        out_type=jax.ShapeDtypeStruct((nrows, D), f32),
        scratch_types=[
            pltpu.VMEM((chunk,), i32),
            pltpu.VMEM((chunk, D), f32),
            pltpu.SemaphoreType.DMA,
        ],
    )
    def k(table_hbm, idx_hbm, out_hbm, idx_v, rows_v, sem):
        wid = jax.lax.axis_index("s") * SC_NC + jax.lax.axis_index("c")
        base = wid * b_per_w
        for ci in range(nch):
            off = base + ci * chunk
            pltpu.sync_copy(idx_hbm.at[pl.ds(off, chunk)], idx_v)
            pltpu.async_copy(table_hbm.at[idx_v], rows_v, sem).wait()
            pltpu.sync_copy(rows_v, out_hbm.at[pl.ds(off, chunk)])

    return k(table, idx)


def _ffn_body(buf_ref, w1_ref, b1_ref, w2_ref, b2_ref, y_ref):
    xb = buf_ref[...].astype(bf16)
    w1 = w1_ref[0].astype(bf16)
    h = jnp.dot(xb, w1, preferred_element_type=f32) + b1_ref[...]
    h = jnp.maximum(h, 0.0)
    w2 = w2_ref[0].astype(bf16)
    y = jnp.dot(h.astype(bf16), w2, preferred_element_type=f32) + b2_ref[...]
    y_ref[...] = y


def _expert_ffn(buf, w1, b1, w2, b2):
    dff = w1.shape[2]
    return pl.pallas_call(
        _ffn_body,
        grid=(E,),
        in_specs=[
            pl.BlockSpec((CAP, D), lambda e: (e, 0)),
            pl.BlockSpec((1, D, dff), lambda e: (e, 0, 0)),
            pl.BlockSpec((1, dff), lambda e: (e, 0)),
            pl.BlockSpec((1, dff, D), lambda e: (e, 0, 0)),
            pl.BlockSpec((1, D), lambda e: (e, 0)),
        ],
        out_specs=pl.BlockSpec((CAP, D), lambda e: (e, 0)),
        out_shape=jax.ShapeDtypeStruct((NSLOT, D), f32),
    )(buf, w1, b1, w2, b2)


def _comb_body(pr_ref, pw_ref, res_ref, g_ref, bb_ref, of_ref, ob_ref):
    r = pr_ref[...]
    w = pw_ref[...]
    moe = r[:, 0, :] * w[:, 0:1] + r[:, 1, :] * w[:, 1:2]
    y = res_ref[...] + moe
    mu = jnp.mean(y, axis=-1, keepdims=True)
    var = jnp.mean((y - mu) ** 2, axis=-1, keepdims=True)
    z = (y - mu) / jnp.sqrt(var + EPS) * g_ref[...] + bb_ref[...]
    of_ref[...] = z
    ob_ref[...] = z.astype(bf16)


def _combine_ln(pair_rows, pair_w, res, g, bb, bm=256):
    return pl.pallas_call(
        _comb_body,
        grid=(S // bm,),
        in_specs=[
            pl.BlockSpec((bm, TOPK, D), lambda i: (i, 0, 0)),
            pl.BlockSpec((bm, TOPK), lambda i: (i, 0)),
            pl.BlockSpec((bm, D), lambda i: (i, 0)),
            pl.BlockSpec((1, D), lambda i: (0, 0)),
            pl.BlockSpec((1, D), lambda i: (0, 0)),
        ],
        out_specs=(
            pl.BlockSpec((bm, D), lambda i: (i, 0)),
            pl.BlockSpec((bm, D), lambda i: (i, 0)),
        ),
        out_shape=(
            jax.ShapeDtypeStruct((S, D), f32),
            jax.ShapeDtypeStruct((S, D), bf16),
        ),
    )(pair_rows, pair_w, res, g.reshape(1, D), bb.reshape(1, D))


def _aux_body(x_ref, w_ref, b_ref, o_ref):
    logits = jnp.dot(x_ref[...], w_ref[...], preferred_element_type=f32)
    logits = logits + b_ref[...]
    mx = jnp.max(logits, axis=-1, keepdims=True)
    ex = jnp.exp(logits - mx)
    probs = ex / jnp.sum(ex, axis=-1, keepdims=True)
    cols = jax.lax.broadcasted_iota(i32, (S, E), 1)
    m1 = jnp.max(probs, axis=-1, keepdims=True)
    i1 = jnp.min(jnp.where(probs == m1, cols, E), axis=-1, keepdims=True)
    probs_m = jnp.where(cols == i1, -jnp.float32(1e30), probs)
    m2 = jnp.max(probs_m, axis=-1, keepdims=True)
    i2 = jnp.min(jnp.where(probs_m == m2, cols, E), axis=-1, keepdims=True)
    oh = (cols == i1).astype(f32) + (cols == i2).astype(f32)
    f = jnp.sum(oh, axis=0, keepdims=True) / float(NPAIR)
    p = jnp.sum(probs, axis=0, keepdims=True) / float(S)
    o_ref[...] = (jnp.sum(f * p) * float(E)).reshape(1, 1)


def _aux_loss(x_bf, rw_bf, rb):
    return pl.pallas_call(
        _aux_body,
        in_specs=[
            pl.BlockSpec((S, D), lambda: (0, 0)),
            pl.BlockSpec((D, E), lambda: (0, 0)),
            pl.BlockSpec((1, E), lambda: (0, 0)),
        ],
        out_specs=pl.BlockSpec((1, 1), lambda: (0, 0)),
        out_shape=jax.ShapeDtypeStruct((1, 1), f32),
    )(x_bf, rw_bf, rb.reshape(1, E))


def _split_heads(x2d):
    return x2d.reshape(S, H, DH).transpose(1, 0, 2)


def _merge_heads(x_hm):
    return x_hm.transpose(1, 0, 2).reshape(S, H * DH)


def kernel(x, encoder_output, sa_Wq, sa_bq, sa_Wk, sa_bk, sa_Wv, sa_bv,
           sa_Wo, sa_bo, ca_Wq, ca_bq, ca_Wk, ca_bk, ca_Wv, ca_bv, ca_Wo,
           ca_bo, ln1_g, ln1_b, ln2_g, ln2_b, ln3_g, ln3_b, router_W,
           router_b, exp_W1, exp_b1, exp_W2, exp_b2):
    x2d = x.reshape(S, D)
    enc2d = encoder_output.reshape(S, D)
    xb = x2d.astype(bf16)
    encb = enc2d.astype(bf16)

    w_sa = jnp.concatenate([sa_Wq, sa_Wk, sa_Wv], axis=1).astype(bf16)
    b_sa = jnp.concatenate([sa_bq, sa_bk, sa_bv])
    qkv = _matmul_bias(xb, w_sa, b_sa, bf16)
    q_hm = _split_heads(qkv[:, :D])
    k_hm = _split_heads(qkv[:, D:2 * D])
    v_hm = _split_heads(qkv[:, 2 * D:])
    a1 = _merge_heads(_flash_attn(q_hm, k_hm, v_hm))
    x1f, x1b = _proj_res_ln(a1, sa_Wo.astype(bf16), sa_bo, x2d, ln1_g, ln1_b)

    q2 = _matmul_bias(x1b, ca_Wq.astype(bf16), ca_bq, bf16)
    w_ckv = jnp.concatenate([ca_Wk, ca_Wv], axis=1).astype(bf16)
    b_ckv = jnp.concatenate([ca_bk, ca_bv])
    kv2 = _matmul_bias(encb, w_ckv, b_ckv, bf16)
    q2_hm = _split_heads(q2)
    k2_hm = _split_heads(kv2[:, :D])
    v2_hm = _split_heads(kv2[:, D:])
    a2 = _merge_heads(_flash_attn(q2_hm, k2_hm, v2_hm))
    x2f, x2b = _proj_res_ln(a2, ca_Wo.astype(bf16), ca_bo, x1f, ln2_g, ln2_b)

    rw_bf = router_W.astype(bf16)
    pair_slot, pair_w, src_tok = _route_dispatch(x2b, rw_bf, router_b)
    buf = _sc_gather(x2f, src_tok.reshape(NSLOT), NSLOT, 80)
    y = _expert_ffn(buf, exp_W1, exp_b1, exp_W2, exp_b2)
    pair_rows = _sc_gather(y, pair_slot.reshape(NPAIR), NPAIR, 64)
    x3f, x3b = _combine_ln(pair_rows.reshape(S, TOPK, D), pair_w, x2f,
                           ln3_g, ln3_b)

    aux = _aux_loss(x3b, rw_bf, router_b)
    return x3f.reshape(1, S, D), aux.reshape(())


# speedup vs baseline: 1.1161x; 1.1161x over previous
import functools
import math

import jax
import jax.numpy as jnp
from jax.experimental import pallas as pl
from jax.experimental.pallas import tpu as pltpu
from jax.experimental.pallas import tpu_sc as plsc

S = 2048
D = 1024
H = 16
DH = 64
E = 64
TOPK = 2
CAP = int(math.ceil(S * TOPK / E * 1.25))
EPS = 1e-5
NSLOT = E * CAP
NPAIR = S * TOPK

f32 = jnp.float32
bf16 = jnp.bfloat16
i32 = jnp.int32

SC_NC = 2
SC_NS = 16
SC_NW = SC_NC * SC_NS


def _mm_body(x_ref, w_ref, b_ref, o_ref):
    acc = jnp.dot(x_ref[...], w_ref[...], preferred_element_type=f32)
    o_ref[...] = (acc + b_ref[...]).astype(o_ref.dtype)


def _matmul_bias(x_bf, w_bf, b, out_dtype, bm=256):
    m, k = x_bf.shape
    n = w_bf.shape[1]
    return pl.pallas_call(
        _mm_body,
        grid=(m // bm,),
        in_specs=[
            pl.BlockSpec((bm, k), lambda i: (i, 0)),
            pl.BlockSpec((k, n), lambda i: (0, 0)),
            pl.BlockSpec((1, n), lambda i: (0, 0)),
        ],
        out_specs=pl.BlockSpec((bm, n), lambda i: (i, 0)),
        out_shape=jax.ShapeDtypeStruct((m, n), out_dtype),
    )(x_bf, w_bf, b.reshape(1, n))


def _attn_body(q_ref, k_ref, v_ref, o_ref):
    q = q_ref[0]
    k = k_ref[0]
    s = jax.lax.dot_general(q, k, (((1,), (1,)), ((), ())),
                            preferred_element_type=f32) * (1.0 / 8.0)
    m = jnp.max(s, axis=-1, keepdims=True)
    p = jnp.exp(s - m)
    p = p / jnp.sum(p, axis=-1, keepdims=True)
    o = jnp.dot(p.astype(bf16), v_ref[0], preferred_element_type=f32)
    o_ref[0] = o.astype(bf16)


def _flash_attn(q_hm, k_hm, v_hm, bq=256):
    return pl.pallas_call(
        _attn_body,
        grid=(H, S // bq),
        in_specs=[
            pl.BlockSpec((1, bq, DH), lambda h, qb: (h, qb, 0)),
            pl.BlockSpec((1, S, DH), lambda h, qb: (h, 0, 0)),
            pl.BlockSpec((1, S, DH), lambda h, qb: (h, 0, 0)),
        ],
        out_specs=pl.BlockSpec((1, bq, DH), lambda h, qb: (h, qb, 0)),
        out_shape=jax.ShapeDtypeStruct((H, S, DH), bf16),
    )(q_hm, k_hm, v_hm)


def _proj_ln_body(a_ref, w_ref, b_ref, res_ref, g_ref, bb_ref, of_ref, ob_ref):
    y = jnp.dot(a_ref[...], w_ref[...], preferred_element_type=f32)
    y = y + b_ref[...] + res_ref[...]
    mu = jnp.mean(y, axis=-1, keepdims=True)
    var = jnp.mean((y - mu) ** 2, axis=-1, keepdims=True)
    z = (y - mu) / jnp.sqrt(var + EPS) * g_ref[...] + bb_ref[...]
    of_ref[...] = z
    ob_ref[...] = z.astype(bf16)


def _proj_res_ln(a_bf, w_bf, b, res, g, bb, bm=256):
    return pl.pallas_call(
        _proj_ln_body,
        grid=(S // bm,),
        in_specs=[
            pl.BlockSpec((bm, D), lambda i: (i, 0)),
            pl.BlockSpec((D, D), lambda i: (0, 0)),
            pl.BlockSpec((1, D), lambda i: (0, 0)),
            pl.BlockSpec((bm, D), lambda i: (i, 0)),
            pl.BlockSpec((1, D), lambda i: (0, 0)),
            pl.BlockSpec((1, D), lambda i: (0, 0)),
        ],
        out_specs=(
            pl.BlockSpec((bm, D), lambda i: (i, 0)),
            pl.BlockSpec((bm, D), lambda i: (i, 0)),
        ),
        out_shape=(
            jax.ShapeDtypeStruct((S, D), f32),
            jax.ShapeDtypeStruct((S, D), bf16),
        ),
    )(a_bf, w_bf, b.reshape(1, D), res, g.reshape(1, D), bb.reshape(1, D))


def _route_body(x_ref, w_ref, b_ref, slot_ref, pw_ref, src_ref):
    logits = jnp.dot(x_ref[...], w_ref[...], preferred_element_type=f32)
    logits = logits + b_ref[...]
    mx = jnp.max(logits, axis=-1, keepdims=True)
    ex = jnp.exp(logits - mx)
    probs = ex / jnp.sum(ex, axis=-1, keepdims=True)

    cols = jax.lax.broadcasted_iota(i32, (S, E), 1)
    m1 = jnp.max(probs, axis=-1, keepdims=True)
    i1 = jnp.min(jnp.where(probs == m1, cols, E), axis=-1, keepdims=True)
    probs_m = jnp.where(cols == i1, -jnp.float32(1e30), probs)
    m2 = jnp.max(probs_m, axis=-1, keepdims=True)
    i2 = jnp.min(jnp.where(probs_m == m2, cols, E), axis=-1, keepdims=True)
    gsum = m1 + m2
    g1 = m1 / gsum
    g2 = m2 / gsum

    ohE = (cols == i1).astype(f32)
    ohO = (cols == i2).astype(f32)

    cb = 256
    r_i = jax.lax.broadcasted_iota(i32, (cb, cb), 0)
    c_i = jax.lax.broadcasted_iota(i32, (cb, cb), 1)
    lt = (r_i >= c_i).astype(f32)
    dnum = (((0,), (0,)), ((), ()))
    pos0_parts = []
    pos1_parts = []
    carryE = jnp.zeros((1, E), f32)
    carryO = jnp.zeros((1, E), f32)
    for ci in range(S // cb):
        sl = slice(ci * cb, (ci + 1) * cb)
        ohE_c = ohE[sl]
        ohO_c = ohO[sl]
        ce = jnp.dot(lt, ohE_c, preferred_element_type=f32) + carryE
        co = jnp.dot(lt, ohO_c, preferred_element_type=f32) + carryO
        tot = ce + co
        pos0_parts.append(
            jnp.sum((tot - ohO_c) * ohE_c, axis=-1, keepdims=True) - 1.0)
        pos1_parts.append(jnp.sum(tot * ohO_c, axis=-1, keepdims=True) - 1.0)
        carryE = ce[cb - 1:cb, :]
        carryO = co[cb - 1:cb, :]
    pos0 = jnp.concatenate(pos0_parts, axis=0)
    pos1 = jnp.concatenate(pos1_parts, axis=0)

    keep0 = (pos0 < CAP).astype(f32)
    keep1 = (pos1 < CAP).astype(f32)
    posc0 = jnp.minimum(pos0, CAP - 1.0)
    posc1 = jnp.minimum(pos1, CAP - 1.0)

    slot_ref[:, 0:1] = i1 * CAP + posc0.astype(i32)
    slot_ref[:, 1:2] = i2 * CAP + posc1.astype(i32)
    pw_ref[:, 0:1] = g1 * keep0
    pw_ref[:, 1:2] = g2 * keep1

    ccap = jax.lax.broadcasted_iota(i32, (S, CAP), 1).astype(f32)
    t_f = jax.lax.broadcasted_iota(i32, (S, 1), 0).astype(f32)
    th = jnp.floor(t_f / 32.0)
    tl = t_f - 32.0 * th
    src = jnp.zeros((E, CAP), f32)
    for oh_k, posc_k, keep_k in ((ohE, posc0, keep0), (ohO, posc1, keep1)):
        ohc = (ccap == posc_k).astype(f32) * keep_k
        src = src + 32.0 * jax.lax.dot_general(
            oh_k, ohc * th, dnum, preferred_element_type=f32)
        src = src + jax.lax.dot_general(
            oh_k, ohc * tl, dnum, preferred_element_type=f32)
    src_ref[...] = src.astype(i32)


def _route_dispatch(x_bf, rw_bf, rb):
    return pl.pallas_call(
        _route_body,
        in_specs=[
            pl.BlockSpec((S, D), lambda: (0, 0)),
            pl.BlockSpec((D, E), lambda: (0, 0)),
            pl.BlockSpec((1, E), lambda: (0, 0)),
        ],
        out_specs=(
            pl.BlockSpec((S, TOPK), lambda: (0, 0)),
            pl.BlockSpec((S, TOPK), lambda: (0, 0)),
            pl.BlockSpec((E, CAP), lambda: (0, 0)),
        ),
        out_shape=(
            jax.ShapeDtypeStruct((S, TOPK), i32),
            jax.ShapeDtypeStruct((S, TOPK), f32),
            jax.ShapeDtypeStruct((E, CAP), i32),
        ),
    )(x_bf, rw_bf, rb.reshape(1, E))


def _sc_gather(table, idx, nrows, chunk):
    b_per_w = nrows // SC_NW
    nch = b_per_w // chunk
    mesh = plsc.VectorSubcoreMesh(core_axis_name="c", subcore_axis_name="s")

    @functools.partial(
        pl.kernel,
        mesh=mesh,
        out_type=jax.ShapeDtypeStruct((nrows, D), f32),
        scratch_types=[
            pltpu.VMEM((chunk,), i32),
            pltpu.VMEM((chunk, D), f32),
            pltpu.SemaphoreType.DMA,
        ],
    )
    def k(table_hbm, idx_hbm, out_hbm, idx_v, rows_v, sem):
        wid = jax.lax.axis_index("s") * SC_NC + jax.lax.axis_index("c")
        base = wid * b_per_w
        for ci in range(nch):
            off = base + ci * chunk
            pltpu.sync_copy(idx_hbm.at[pl.ds(off, chunk)], idx_v)
            pltpu.async_copy(table_hbm.at[idx_v], rows_v, sem).wait()
            pltpu.sync_copy(rows_v, out_hbm.at[pl.ds(off, chunk)])

    return k(table, idx)


def _ffn_body(buf_ref, w1_ref, b1_ref, w2_ref, b2_ref, y_ref):
    xb = buf_ref[...].astype(bf16)
    w1 = w1_ref[0].astype(bf16)
    h = jnp.dot(xb, w1, preferred_element_type=f32) + b1_ref[0]
    h = jnp.maximum(h, 0.0)
    w2 = w2_ref[0].astype(bf16)
    y = jnp.dot(h.astype(bf16), w2, preferred_element_type=f32) + b2_ref[0]
    y_ref[...] = y


def _expert_ffn(buf, w1, b1, w2, b2):
    dff = w1.shape[2]
    return pl.pallas_call(
        _ffn_body,
        grid=(E,),
        in_specs=[
            pl.BlockSpec((CAP, D), lambda e: (e, 0)),
            pl.BlockSpec((1, D, dff), lambda e: (e, 0, 0)),
            pl.BlockSpec((1, 1, dff), lambda e: (e, 0, 0)),
            pl.BlockSpec((1, dff, D), lambda e: (e, 0, 0)),
            pl.BlockSpec((1, 1, D), lambda e: (e, 0, 0)),
        ],
        out_specs=pl.BlockSpec((CAP, D), lambda e: (e, 0)),
        out_shape=jax.ShapeDtypeStruct((NSLOT, D), f32),
    )(buf, w1, b1.reshape(E, 1, dff), w2, b2.reshape(E, 1, D))


def _comb_body(pr_ref, pw_ref, res_ref, g_ref, bb_ref, of_ref, ob_ref):
    r = pr_ref[...]
    w = pw_ref[...]
    moe = r[:, 0, :] * w[:, 0:1] + r[:, 1, :] * w[:, 1:2]
    y = res_ref[...] + moe
    mu = jnp.mean(y, axis=-1, keepdims=True)
    var = jnp.mean((y - mu) ** 2, axis=-1, keepdims=True)
    z = (y - mu) / jnp.sqrt(var + EPS) * g_ref[...] + bb_ref[...]
    of_ref[...] = z
    ob_ref[...] = z.astype(bf16)


def _combine_ln(pair_rows, pair_w, res, g, bb, bm=256):
    return pl.pallas_call(
        _comb_body,
        grid=(S // bm,),
        in_specs=[
            pl.BlockSpec((bm, TOPK, D), lambda i: (i, 0, 0)),
            pl.BlockSpec((bm, TOPK), lambda i: (i, 0)),
            pl.BlockSpec((bm, D), lambda i: (i, 0)),
            pl.BlockSpec((1, D), lambda i: (0, 0)),
            pl.BlockSpec((1, D), lambda i: (0, 0)),
        ],
        out_specs=(
            pl.BlockSpec((bm, D), lambda i: (i, 0)),
            pl.BlockSpec((bm, D), lambda i: (i, 0)),
        ),
        out_shape=(
            jax.ShapeDtypeStruct((S, D), f32),
            jax.ShapeDtypeStruct((S, D), bf16),
        ),
    )(pair_rows, pair_w, res, g.reshape(1, D), bb.reshape(1, D))


def _aux_body(x_ref, w_ref, b_ref, o_ref):
    logits = jnp.dot(x_ref[...], w_ref[...], preferred_element_type=f32)
    logits = logits + b_ref[...]
    mx = jnp.max(logits, axis=-1, keepdims=True)
    ex = jnp.exp(logits - mx)
    probs = ex / jnp.sum(ex, axis=-1, keepdims=True)
    cols = jax.lax.broadcasted_iota(i32, (S, E), 1)
    m1 = jnp.max(probs, axis=-1, keepdims=True)
    i1 = jnp.min(jnp.where(probs == m1, cols, E), axis=-1, keepdims=True)
    probs_m = jnp.where(cols == i1, -jnp.float32(1e30), probs)
    m2 = jnp.max(probs_m, axis=-1, keepdims=True)
    i2 = jnp.min(jnp.where(probs_m == m2, cols, E), axis=-1, keepdims=True)
    oh = (cols == i1).astype(f32) + (cols == i2).astype(f32)
    f = jnp.sum(oh, axis=0, keepdims=True) / float(NPAIR)
    p = jnp.sum(probs, axis=0, keepdims=True) / float(S)
    o_ref[...] = (jnp.sum(f * p) * float(E)).reshape(1, 1)


def _aux_loss(x_bf, rw_bf, rb):
    return pl.pallas_call(
        _aux_body,
        in_specs=[
            pl.BlockSpec((S, D), lambda: (0, 0)),
            pl.BlockSpec((D, E), lambda: (0, 0)),
            pl.BlockSpec((1, E), lambda: (0, 0)),
        ],
        out_specs=pl.BlockSpec((1, 1), lambda: (0, 0)),
        out_shape=jax.ShapeDtypeStruct((1, 1), f32),
    )(x_bf, rw_bf, rb.reshape(1, E))


def _split_heads(x2d):
    return x2d.reshape(S, H, DH).transpose(1, 0, 2)


def _merge_heads(x_hm):
    return x_hm.transpose(1, 0, 2).reshape(S, H * DH)


def kernel(x, encoder_output, sa_Wq, sa_bq, sa_Wk, sa_bk, sa_Wv, sa_bv,
           sa_Wo, sa_bo, ca_Wq, ca_bq, ca_Wk, ca_bk, ca_Wv, ca_bv, ca_Wo,
           ca_bo, ln1_g, ln1_b, ln2_g, ln2_b, ln3_g, ln3_b, router_W,
           router_b, exp_W1, exp_b1, exp_W2, exp_b2):
    x2d = x.reshape(S, D)
    enc2d = encoder_output.reshape(S, D)
    xb = x2d.astype(bf16)
    encb = enc2d.astype(bf16)

    w_sa = jnp.concatenate([sa_Wq, sa_Wk, sa_Wv], axis=1).astype(bf16)
    b_sa = jnp.concatenate([sa_bq, sa_bk, sa_bv])
    qkv = _matmul_bias(xb, w_sa, b_sa, bf16)
    q_hm = _split_heads(qkv[:, :D])
    k_hm = _split_heads(qkv[:, D:2 * D])
    v_hm = _split_heads(qkv[:, 2 * D:])
    a1 = _merge_heads(_flash_attn(q_hm, k_hm, v_hm))
    x1f, x1b = _proj_res_ln(a1, sa_Wo.astype(bf16), sa_bo, x2d, ln1_g, ln1_b)

    q2 = _matmul_bias(x1b, ca_Wq.astype(bf16), ca_bq, bf16)
    w_ckv = jnp.concatenate([ca_Wk, ca_Wv], axis=1).astype(bf16)
    b_ckv = jnp.concatenate([ca_bk, ca_bv])
    kv2 = _matmul_bias(encb, w_ckv, b_ckv, bf16)
    q2_hm = _split_heads(q2)
    k2_hm = _split_heads(kv2[:, :D])
    v2_hm = _split_heads(kv2[:, D:])
    a2 = _merge_heads(_flash_attn(q2_hm, k2_hm, v2_hm))
    x2f, x2b = _proj_res_ln(a2, ca_Wo.astype(bf16), ca_bo, x1f, ln2_g, ln2_b)

    rw_bf = router_W.astype(bf16)
    pair_slot, pair_w, src_tok = _route_dispatch(x2b, rw_bf, router_b)
    buf = _sc_gather(x2f, src_tok.reshape(NSLOT), NSLOT, 80)
    y = _expert_ffn(buf, exp_W1, exp_b1, exp_W2, exp_b2)
    pair_rows = _sc_gather(y, pair_slot.reshape(NPAIR), NPAIR, 64)
    x3f, x3b = _combine_ln(pair_rows.reshape(S, TOPK, D), pair_w, x2f,
                           ln3_g, ln3_b)

    aux = _aux_loss(x3b, rw_bf, router_b)
    return x3f.reshape(1, S, D), aux.reshape(())


# speedup vs baseline: 1.3013x; 1.1659x over previous
import functools
import math

import jax
import jax.numpy as jnp
from jax.experimental import pallas as pl
from jax.experimental.pallas import tpu as pltpu
from jax.experimental.pallas import tpu_sc as plsc

S = 2048
D = 1024
H = 16
DH = 64
E = 64
TOPK = 2
CAP = int(math.ceil(S * TOPK / E * 1.25))
EPS = 1e-5
NSLOT = E * CAP
NPAIR = S * TOPK

f32 = jnp.float32
bf16 = jnp.bfloat16
i32 = jnp.int32

SC_NC = 2
SC_NS = 16
SC_NW = SC_NC * SC_NS


def _mm_body(x_ref, w_ref, b_ref, o_ref):
    acc = jnp.dot(x_ref[...], w_ref[...], preferred_element_type=f32)
    o_ref[...] = (acc + b_ref[...]).astype(o_ref.dtype)


def _matmul_bias(x_bf, w_bf, b, out_dtype, bm=256):
    m, k = x_bf.shape
    n = w_bf.shape[1]
    return pl.pallas_call(
        _mm_body,
        grid=(m // bm,),
        in_specs=[
            pl.BlockSpec((bm, k), lambda i: (i, 0)),
            pl.BlockSpec((k, n), lambda i: (0, 0)),
            pl.BlockSpec((1, n), lambda i: (0, 0)),
        ],
        out_specs=pl.BlockSpec((bm, n), lambda i: (i, 0)),
        out_shape=jax.ShapeDtypeStruct((m, n), out_dtype),
    )(x_bf, w_bf, b.reshape(1, n))


def _attn_body(q_ref, k_ref, v_ref, o_ref):
    qq = q_ref[...]
    kk = k_ref[...]
    outs = []
    for h in range(2):
        q = qq[:, h * DH:(h + 1) * DH]
        k = kk[:, h * DH:(h + 1) * DH]
        s = jax.lax.dot_general(q, k, (((1,), (1,)), ((), ())),
                                preferred_element_type=f32) * (1.0 / 8.0)
        m = jnp.max(s, axis=-1, keepdims=True)
        p = jnp.exp(s - m)
        p = p / jnp.sum(p, axis=-1, keepdims=True)
        v = v_ref[:, h * DH:(h + 1) * DH]
        outs.append(jnp.dot(p.astype(bf16), v, preferred_element_type=f32))
    o_ref[...] = jnp.concatenate(outs, axis=1).astype(bf16)


def _flash_attn(q_arr, kv_arr, q_cb, k_cb, v_cb, bq=256):
    nhp = H // 2
    return pl.pallas_call(
        _attn_body,
        grid=(nhp, S // bq),
        in_specs=[
            pl.BlockSpec((bq, 2 * DH), lambda hp, qb, o=q_cb: (qb, o + hp)),
            pl.BlockSpec((S, 2 * DH), lambda hp, qb, o=k_cb: (0, o + hp)),
            pl.BlockSpec((S, 2 * DH), lambda hp, qb, o=v_cb: (0, o + hp)),
        ],
        out_specs=pl.BlockSpec((bq, 2 * DH), lambda hp, qb: (qb, hp)),
        out_shape=jax.ShapeDtypeStruct((S, D), bf16),
    )(q_arr, kv_arr, kv_arr)


def _proj_ln_body(a_ref, w_ref, b_ref, res_ref, g_ref, bb_ref, of_ref, ob_ref):
    y = jnp.dot(a_ref[...], w_ref[...], preferred_element_type=f32)
    y = y + b_ref[...] + res_ref[...]
    mu = jnp.mean(y, axis=-1, keepdims=True)
    var = jnp.mean((y - mu) ** 2, axis=-1, keepdims=True)
    z = (y - mu) / jnp.sqrt(var + EPS) * g_ref[...] + bb_ref[...]
    of_ref[...] = z
    ob_ref[...] = z.astype(bf16)


def _proj_res_ln(a_bf, w_bf, b, res, g, bb, bm=256):
    return pl.pallas_call(
        _proj_ln_body,
        grid=(S // bm,),
        in_specs=[
            pl.BlockSpec((bm, D), lambda i: (i, 0)),
            pl.BlockSpec((D, D), lambda i: (0, 0)),
            pl.BlockSpec((1, D), lambda i: (0, 0)),
            pl.BlockSpec((bm, D), lambda i: (i, 0)),
            pl.BlockSpec((1, D), lambda i: (0, 0)),
            pl.BlockSpec((1, D), lambda i: (0, 0)),
        ],
        out_specs=(
            pl.BlockSpec((bm, D), lambda i: (i, 0)),
            pl.BlockSpec((bm, D), lambda i: (i, 0)),
        ),
        out_shape=(
            jax.ShapeDtypeStruct((S, D), f32),
            jax.ShapeDtypeStruct((S, D), bf16),
        ),
    )(a_bf, w_bf, b.reshape(1, D), res, g.reshape(1, D), bb.reshape(1, D))


def _route_body(x_ref, w_ref, b_ref, slot_ref, pw_ref, src_ref):
    logits = jnp.dot(x_ref[...], w_ref[...], preferred_element_type=f32)
    logits = logits + b_ref[...]
    mx = jnp.max(logits, axis=-1, keepdims=True)
    ex = jnp.exp(logits - mx)
    probs = ex / jnp.sum(ex, axis=-1, keepdims=True)

    cols = jax.lax.broadcasted_iota(i32, (S, E), 1)
    m1 = jnp.max(probs, axis=-1, keepdims=True)
    i1 = jnp.min(jnp.where(probs == m1, cols, E), axis=-1, keepdims=True)
    probs_m = jnp.where(cols == i1, -jnp.float32(1e30), probs)
    m2 = jnp.max(probs_m, axis=-1, keepdims=True)
    i2 = jnp.min(jnp.where(probs_m == m2, cols, E), axis=-1, keepdims=True)
    gsum = m1 + m2
    g1 = m1 / gsum
    g2 = m2 / gsum

    ohE = (cols == i1).astype(f32)
    ohO = (cols == i2).astype(f32)

    cb = 256
    r_i = jax.lax.broadcasted_iota(i32, (cb, cb), 0)
    c_i = jax.lax.broadcasted_iota(i32, (cb, cb), 1)
    lt = (r_i >= c_i).astype(f32)
    dnum = (((0,), (0,)), ((), ()))
    pos0_parts = []
    pos1_parts = []
    carryE = jnp.zeros((1, E), f32)
    carryO = jnp.zeros((1, E), f32)
    for ci in range(S // cb):
        sl = slice(ci * cb, (ci + 1) * cb)
        ohE_c = ohE[sl]
        ohO_c = ohO[sl]
        ce = jnp.dot(lt, ohE_c, preferred_element_type=f32) + carryE
        co = jnp.dot(lt, ohO_c, preferred_element_type=f32) + carryO
        tot = ce + co
        pos0_parts.append(
            jnp.sum((tot - ohO_c) * ohE_c, axis=-1, keepdims=True) - 1.0)
        pos1_parts.append(jnp.sum(tot * ohO_c, axis=-1, keepdims=True) - 1.0)
        carryE = ce[cb - 1:cb, :]
        carryO = co[cb - 1:cb, :]
    pos0 = jnp.concatenate(pos0_parts, axis=0)
    pos1 = jnp.concatenate(pos1_parts, axis=0)

    keep0 = (pos0 < CAP).astype(f32)
    keep1 = (pos1 < CAP).astype(f32)
    posc0 = jnp.minimum(pos0, CAP - 1.0)
    posc1 = jnp.minimum(pos1, CAP - 1.0)

    slot_ref[:, 0:1] = i1 * CAP + posc0.astype(i32)
    slot_ref[:, 1:2] = i2 * CAP + posc1.astype(i32)
    pw_ref[:, 0:1] = g1 * keep0
    pw_ref[:, 1:2] = g2 * keep1

    ccap = jax.lax.broadcasted_iota(i32, (S, CAP), 1).astype(f32)
    t_f = jax.lax.broadcasted_iota(i32, (S, 1), 0).astype(f32)
    th = jnp.floor(t_f / 32.0)
    tl = t_f - 32.0 * th
    src = jnp.zeros((E, CAP), f32)
    for oh_k, posc_k, keep_k in ((ohE, posc0, keep0), (ohO, posc1, keep1)):
        ohc = (ccap == posc_k).astype(f32) * keep_k
        src = src + 32.0 * jax.lax.dot_general(
            oh_k, ohc * th, dnum, preferred_element_type=f32)
        src = src + jax.lax.dot_general(
            oh_k, ohc * tl, dnum, preferred_element_type=f32)
    src_ref[...] = src.astype(i32)


def _route_dispatch(x_bf, rw_bf, rb):
    return pl.pallas_call(
        _route_body,
        in_specs=[
            pl.BlockSpec((S, D), lambda: (0, 0)),
            pl.BlockSpec((D, E), lambda: (0, 0)),
            pl.BlockSpec((1, E), lambda: (0, 0)),
        ],
        out_specs=(
            pl.BlockSpec((S, TOPK), lambda: (0, 0)),
            pl.BlockSpec((S, TOPK), lambda: (0, 0)),
            pl.BlockSpec((E, CAP), lambda: (0, 0)),
        ),
        out_shape=(
            jax.ShapeDtypeStruct((S, TOPK), i32),
            jax.ShapeDtypeStruct((S, TOPK), f32),
            jax.ShapeDtypeStruct((E, CAP), i32),
        ),
    )(x_bf, rw_bf, rb.reshape(1, E))


def _sc_gather(table, idx, nrows, chunk):
    b_per_w = nrows // SC_NW
    nch = b_per_w // chunk
    mesh = plsc.VectorSubcoreMesh(core_axis_name="c", subcore_axis_name="s")

    @functools.partial(
        pl.kernel,
        mesh=mesh,
        out_type=jax.ShapeDtypeStruct((nrows, D), f32),
        scratch_types=[
            pltpu.VMEM((chunk,), i32),
            pltpu.VMEM((chunk, D), f32),
            pltpu.SemaphoreType.DMA,
        ],
    )
    def k(table_hbm, idx_hbm, out_hbm, idx_v, rows_v, sem):
        wid = jax.lax.axis_index("s") * SC_NC + jax.lax.axis_index("c")
        base = wid * b_per_w
        for ci in range(nch):
            off = base + ci * chunk
            pltpu.sync_copy(idx_hbm.at[pl.ds(off, chunk)], idx_v)
            pltpu.async_copy(table_hbm.at[idx_v], rows_v, sem).wait()
            pltpu.sync_copy(rows_v, out_hbm.at[pl.ds(off, chunk)])

    return k(table, idx)


def _ffn_body(buf_ref, w1_ref, b1_ref, w2_ref, b2_ref, y_ref):
    xb = buf_ref[...].astype(bf16)
    w1 = w1_ref[0].astype(bf16)
    h = jnp.dot(xb, w1, preferred_element_type=f32) + b1_ref[0]
    h = jnp.maximum(h, 0.0)
    w2 = w2_ref[0].astype(bf16)
    y = jnp.dot(h.astype(bf16), w2, preferred_element_type=f32) + b2_ref[0]
    y_ref[...] = y


def _expert_ffn(buf, w1, b1, w2, b2):
    dff = w1.shape[2]
    return pl.pallas_call(
        _ffn_body,
        grid=(E,),
        in_specs=[
            pl.BlockSpec((CAP, D), lambda e: (e, 0)),
            pl.BlockSpec((1, D, dff), lambda e: (e, 0, 0)),
            pl.BlockSpec((1, 1, dff), lambda e: (e, 0, 0)),
            pl.BlockSpec((1, dff, D), lambda e: (e, 0, 0)),
            pl.BlockSpec((1, 1, D), lambda e: (e, 0, 0)),
        ],
        out_specs=pl.BlockSpec((CAP, D), lambda e: (e, 0)),
        out_shape=jax.ShapeDtypeStruct((NSLOT, D), f32),
    )(buf, w1, b1.reshape(E, 1, dff), w2, b2.reshape(E, 1, D))


def _comb_body(pr_ref, pw_ref, res_ref, g_ref, bb_ref, of_ref, ob_ref):
    r = pr_ref[...]
    w = pw_ref[...]
    moe = r[:, 0, :] * w[:, 0:1] + r[:, 1, :] * w[:, 1:2]
    y = res_ref[...] + moe
    mu = jnp.mean(y, axis=-1, keepdims=True)
    var = jnp.mean((y - mu) ** 2, axis=-1, keepdims=True)
    z = (y - mu) / jnp.sqrt(var + EPS) * g_ref[...] + bb_ref[...]
    of_ref[...] = z
    ob_ref[...] = z.astype(bf16)


def _combine_ln(pair_rows, pair_w, res, g, bb, bm=256):
    return pl.pallas_call(
        _comb_body,
        grid=(S // bm,),
        in_specs=[
            pl.BlockSpec((bm, TOPK, D), lambda i: (i, 0, 0)),
            pl.BlockSpec((bm, TOPK), lambda i: (i, 0)),
            pl.BlockSpec((bm, D), lambda i: (i, 0)),
            pl.BlockSpec((1, D), lambda i: (0, 0)),
            pl.BlockSpec((1, D), lambda i: (0, 0)),
        ],
        out_specs=(
            pl.BlockSpec((bm, D), lambda i: (i, 0)),
            pl.BlockSpec((bm, D), lambda i: (i, 0)),
        ),
        out_shape=(
            jax.ShapeDtypeStruct((S, D), f32),
            jax.ShapeDtypeStruct((S, D), bf16),
        ),
    )(pair_rows, pair_w, res, g.reshape(1, D), bb.reshape(1, D))


def _aux_body(x_ref, w_ref, b_ref, o_ref):
    logits = jnp.dot(x_ref[...], w_ref[...], preferred_element_type=f32)
    logits = logits + b_ref[...]
    mx = jnp.max(logits, axis=-1, keepdims=True)
    ex = jnp.exp(logits - mx)
    probs = ex / jnp.sum(ex, axis=-1, keepdims=True)
    cols = jax.lax.broadcasted_iota(i32, (S, E), 1)
    m1 = jnp.max(probs, axis=-1, keepdims=True)
    i1 = jnp.min(jnp.where(probs == m1, cols, E), axis=-1, keepdims=True)
    probs_m = jnp.where(cols == i1, -jnp.float32(1e30), probs)
    m2 = jnp.max(probs_m, axis=-1, keepdims=True)
    i2 = jnp.min(jnp.where(probs_m == m2, cols, E), axis=-1, keepdims=True)
    oh = (cols == i1).astype(f32) + (cols == i2).astype(f32)
    f = jnp.sum(oh, axis=0, keepdims=True) / float(NPAIR)
    p = jnp.sum(probs, axis=0, keepdims=True) / float(S)
    o_ref[...] = (jnp.sum(f * p) * float(E)).reshape(1, 1)


def _aux_loss(x_bf, rw_bf, rb):
    return pl.pallas_call(
        _aux_body,
        in_specs=[
            pl.BlockSpec((S, D), lambda: (0, 0)),
            pl.BlockSpec((D, E), lambda: (0, 0)),
            pl.BlockSpec((1, E), lambda: (0, 0)),
        ],
        out_specs=pl.BlockSpec((1, 1), lambda: (0, 0)),
        out_shape=jax.ShapeDtypeStruct((1, 1), f32),
    )(x_bf, rw_bf, rb.reshape(1, E))


def kernel(x, encoder_output, sa_Wq, sa_bq, sa_Wk, sa_bk, sa_Wv, sa_bv,
           sa_Wo, sa_bo, ca_Wq, ca_bq, ca_Wk, ca_bk, ca_Wv, ca_bv, ca_Wo,
           ca_bo, ln1_g, ln1_b, ln2_g, ln2_b, ln3_g, ln3_b, router_W,
           router_b, exp_W1, exp_b1, exp_W2, exp_b2):
    x2d = x.reshape(S, D)
    enc2d = encoder_output.reshape(S, D)
    xb = x2d.astype(bf16)
    encb = enc2d.astype(bf16)

    w_sa = jnp.concatenate([sa_Wq, sa_Wk, sa_Wv], axis=1).astype(bf16)
    b_sa = jnp.concatenate([sa_bq, sa_bk, sa_bv])
    qkv = _matmul_bias(xb, w_sa, b_sa, bf16)
    a1 = _flash_attn(qkv, qkv, 0, 8, 16)
    x1f, x1b = _proj_res_ln(a1, sa_Wo.astype(bf16), sa_bo, x2d, ln1_g, ln1_b)

    q2 = _matmul_bias(x1b, ca_Wq.astype(bf16), ca_bq, bf16)
    w_ckv = jnp.concatenate([ca_Wk, ca_Wv], axis=1).astype(bf16)
    b_ckv = jnp.concatenate([ca_bk, ca_bv])
    kv2 = _matmul_bias(encb, w_ckv, b_ckv, bf16)
    a2 = _flash_attn(q2, kv2, 0, 0, 8)
    x2f, x2b = _proj_res_ln(a2, ca_Wo.astype(bf16), ca_bo, x1f, ln2_g, ln2_b)

    rw_bf = router_W.astype(bf16)
    pair_slot, pair_w, src_tok = _route_dispatch(x2b, rw_bf, router_b)
    buf = _sc_gather(x2f, src_tok.reshape(NSLOT), NSLOT, 80)
    y = _expert_ffn(buf, exp_W1, exp_b1, exp_W2, exp_b2)
    pair_rows = _sc_gather(y, pair_slot.reshape(NPAIR), NPAIR, 64)
    x3f, x3b = _combine_ln(pair_rows.reshape(S, TOPK, D), pair_w, x2f,
                           ln3_g, ln3_b)

    aux = _aux_loss(x3b, rw_bf, router_b)
    return x3f.reshape(1, S, D), aux.reshape(())
